# baseline (speedup 1.0000x reference)
import functools

import jax
import jax.numpy as jnp
from jax import lax
from jax.experimental import pallas as pl
from jax.experimental.pallas import tpu as pltpu
from jax.experimental.pallas import tpu_sc as plsc

V = 1_000_000
D = 64
VP = V // 2
NC, NS = 2, 16
NW = NC * NS
B = 4096 * 200
NBLK = B // 128
BLK_PER_W = NBLK // NW
FULL_COLS = (V // 128) * 128
NFULL = FULL_COLS // 128
RPW = NFULL // NW


def _iota16():
    return lax.iota(jnp.int32, 16)


def _repack_block(in_v, out_v, nrows):
    def row(r, carry):
        for g in range(8):
            h = g // 4
            d_vec = _iota16() + 16 * (g % 4)
            col = jnp.full((16,), 1, jnp.int32) * (2 * r + h)
            vals = plsc.load_gather(in_v, [d_vec, col])
            out_v[r, pl.ds(16 * g, 16)] = vals
        return carry

    lax.fori_loop(0, nrows, row, 0)


def _build_repack():
    mesh = plsc.VectorSubcoreMesh(core_axis_name="c", subcore_axis_name="s")

    @functools.partial(
        pl.kernel,
        mesh=mesh,
        out_type=jax.ShapeDtypeStruct((VP, 128), jnp.float32),
        scratch_types=[
            pltpu.VMEM((D, 128), jnp.float32),
            pltpu.VMEM((D, 128), jnp.float32),
            pltpu.VMEM((D, 64), jnp.float32),
            pltpu.VMEM((D, 128), jnp.float32),
            pltpu.VMEM((D, 128), jnp.float32),
            pltpu.SemaphoreType.DMA,
            pltpu.SemaphoreType.DMA,
            pltpu.SemaphoreType.DMA,
            pltpu.SemaphoreType.DMA,
        ],
        compiler_params=pltpu.CompilerParams(use_tc_tiling_on_sc=True, needs_layout_passes=False),
    )
    def repack(tT_hbm, r_hbm, in0, in1, int_, o0, o1, si0, si1, so0, so1):
        wid = lax.axis_index("s") * NC + lax.axis_index("c")
        m0 = wid * RPW

        def in_desc(m, buf, sem):
            return pltpu.make_async_copy(
                tT_hbm.at[:, pl.ds(m * 128, 128)], buf, sem
            )

        def out_desc(m, buf, sem):
            return pltpu.make_async_copy(
                buf, r_hbm.at[pl.ds(m * 64, D), :], sem
            )

        in_desc(m0, in0, si0).start()
        in_desc(m0 + 1, in1, si1).start()

        def body(u, carry):
            ma = m0 + 2 * u
            for (mb, in_v, out_v, si, so) in (
                (ma, in0, o0, si0, so0),
                (ma + 1, in1, o1, si1, so1),
            ):
                in_desc(mb, in_v, si).wait()
                pl.when(u > 0)(lambda: out_desc(mb - 2, out_v, so).wait())
                _repack_block(in_v, out_v, D)
                out_desc(mb, out_v, so).start()
                pl.when(u < RPW // 2 - 1)(
                    lambda: in_desc(mb + 2, in_v, si).start()
                )
            return carry

        lax.fori_loop(0, RPW // 2, body, 0)
        out_desc(m0 + RPW - 2, o0, so0).wait()
        out_desc(m0 + RPW - 1, o1, so1).wait()

        @pl.when(wid < 4)
        def _extra():
            m = NFULL - 4 + wid
            in_desc(m, in0, si0).start()
            in_desc(m, in0, si0).wait()
            _repack_block(in0, o0, D)
            out_desc(m, o0, so0).start()
            out_desc(m, o0, so0).wait()

        @pl.when(wid == NW - 1)
        def _tail():
            tin = pltpu.make_async_copy(
                tT_hbm.at[:, pl.ds(FULL_COLS, 64)], int_, si1
            )
            tin.start()
            tin.wait()
            _repack_block(int_, o1, 32)
            tout = pltpu.make_async_copy(
                o1.at[pl.ds(0, 32), :],
                r_hbm.at[pl.ds(FULL_COLS // 2, 32), :],
                so1,
            )
            tout.start()
            tout.wait()

    return repack


def _build_gather():
    mesh = plsc.VectorSubcoreMesh(core_axis_name="c", subcore_axis_name="s")

    @functools.partial(
        pl.kernel,
        mesh=mesh,
        out_type=jax.ShapeDtypeStruct((200, D, 4096), jnp.float32),
        scratch_types=[
            pltpu.VMEM((128,), jnp.int32),
            pltpu.VMEM((128,), jnp.int32),
            pltpu.VMEM((128,), jnp.int32),
            pltpu.VMEM((128,), jnp.int32),
            pltpu.VMEM((128,), jnp.int32),
            pltpu.VMEM((128,), jnp.int32),
            pltpu.VMEM((128, 128), jnp.float32),
            pltpu.VMEM((128, 128), jnp.float32),
            pltpu.VMEM((D, 128), jnp.float32),
            pltpu.VMEM((D, 128), jnp.float32),
            pltpu.SemaphoreType.DMA,
            pltpu.SemaphoreType.DMA,
            pltpu.SemaphoreType.DMA,
            pltpu.SemaphoreType.DMA,
            pltpu.SemaphoreType.DMA,
            pltpu.SemaphoreType.DMA,
        ],
        compiler_params=pltpu.CompilerParams(use_tc_tiling_on_sc=True, needs_layout_passes=False),
    )
    def gather(
        xf_hbm, r_hbm, out_hbm,
        ix0, ix1, aj0, aj1, hf0, hf1, rw0, rw1, sl0, sl1,
        sx0, sx1, sg0, sg1, so0, so1,
    ):
        wid = lax.axis_index("s") * NC + lax.axis_index("c")
        b0 = wid * BLK_PER_W

        def idx_desc(b, buf, sem):
            return pltpu.make_async_copy(
                xf_hbm.at[pl.ds(b * 128, 128)], buf, sem
            )

        def gat_desc(adj, buf, sem):
            return pltpu.make_async_copy(r_hbm.at[adj], buf, sem)

        def out_desc(b, buf, sem):
            j = lax.shift_right_logical(b, 5)
            i_hi = lax.bitwise_and(b, 31)
            return pltpu.make_async_copy(
                buf, out_hbm.at[j, :, pl.ds(i_hi * 128, 128)], sem
            )

        def arith(ix, adj, hf):
            for k in range(8):
                v = ix[pl.ds(16 * k, 16)]
                adj[pl.ds(16 * k, 16)] = lax.shift_right_logical(v, 1)
                hf[pl.ds(16 * k, 16)] = lax.bitwise_and(v, 1)

        def transpose(rw, hf, sl):
            def row(d, carry):
                for g in range(8):
                    i_vec = _iota16() + 16 * g
                    h_vec = hf[pl.ds(16 * g, 16)]
                    vals = plsc.load_gather(rw, [i_vec, h_vec * 64 + d])
                    sl[d, pl.ds(16 * g, 16)] = vals
                return carry

            lax.fori_loop(0, D, row, 0)

        idx_desc(b0, ix0, sx0).start()
        idx_desc(b0, ix0, sx0).wait()
        arith(ix0, aj0, hf0)
        gat_desc(aj0, rw0, sg0).start()
        idx_desc(b0 + 1, ix1, sx1).start()
        idx_desc(b0 + 1, ix1, sx1).wait()
        arith(ix1, aj1, hf1)
        gat_desc(aj1, rw1, sg1).start()

        def body(u, carry):
            ba = b0 + 2 * u
            for (bb, ix, aj, hf, rw, sl, sx, sg, so) in (
                (ba, ix0, aj0, hf0, rw0, sl0, sx0, sg0, so0),
                (ba + 1, ix1, aj1, hf1, rw1, sl1, sx1, sg1, so1),
            ):
                gat_desc(aj, rw, sg).wait()
                pl.when(u > 0)(lambda: out_desc(bb - 2, sl, so).wait())
                transpose(rw, hf, sl)
                out_desc(bb, sl, so).start()

                @pl.when(u < BLK_PER_W // 2 - 1)
                def _next():
                    idx_desc(bb + 2, ix, sx).start()
                    idx_desc(bb + 2, ix, sx).wait()
                    arith(ix, aj, hf)
                    gat_desc(aj, rw, sg).start()

            return carry

        lax.fori_loop(0, BLK_PER_W // 2, body, 0)
        out_desc(b0 + BLK_PER_W - 2, sl0, so0).wait()
        out_desc(b0 + BLK_PER_W - 1, sl1, so1).wait()

    return gather


def kernel(x, table):
    xf = x.T.reshape(B).astype(jnp.int32)
    tT = table.T
    repacked = _build_repack()(tT)
    out2 = _build_gather()(xf, repacked)
    return out2.transpose(2, 0, 1)


# speedup vs baseline: 2.2677x; 2.2677x over previous
import functools

import jax
import jax.numpy as jnp
from jax import lax
from jax.experimental import pallas as pl
from jax.experimental.pallas import tpu as pltpu
from jax.experimental.pallas import tpu_sc as plsc

V = 1_000_000
D = 64
VP = V // 2
NC, NS = 2, 16
NW = NC * NS
B = 4096 * 200
NBLK = B // 128
BLK_PER_W = NBLK // NW
FULL_COLS = (V // 128) * 128
NFULL = FULL_COLS // 128
RPW = NFULL // NW


def _iota16():
    return lax.iota(jnp.int32, 16)


def _repack_block(in_v, out_v, nrows):
    d_vecs = [_iota16() + 16 * q for q in range(4)]

    @plsc.parallel_loop(0, nrows, unroll=8)
    def row(r):
        for g in range(8):
            h = g // 4
            col = jnp.full((16,), 1, jnp.int32) * (2 * r + h)
            vals = plsc.load_gather(in_v, [d_vecs[g % 4], col])
            out_v[r, pl.ds(16 * g, 16)] = vals


def _build_repack():
    mesh = plsc.VectorSubcoreMesh(core_axis_name="c", subcore_axis_name="s")

    @functools.partial(
        pl.kernel,
        mesh=mesh,
        out_type=jax.ShapeDtypeStruct((VP, 128), jnp.float32),
        scratch_types=[
            pltpu.VMEM((D, 128), jnp.float32),
            pltpu.VMEM((D, 128), jnp.float32),
            pltpu.VMEM((D, 64), jnp.float32),
            pltpu.VMEM((D, 128), jnp.float32),
            pltpu.VMEM((D, 128), jnp.float32),
            pltpu.SemaphoreType.DMA,
            pltpu.SemaphoreType.DMA,
            pltpu.SemaphoreType.DMA,
            pltpu.SemaphoreType.DMA,
        ],
        compiler_params=pltpu.CompilerParams(use_tc_tiling_on_sc=True, needs_layout_passes=False),
    )
    def repack(tT_hbm, r_hbm, in0, in1, int_, o0, o1, si0, si1, so0, so1):
        wid = lax.axis_index("s") * NC + lax.axis_index("c")
        m0 = wid * RPW

        def in_desc(m, buf, sem):
            return pltpu.make_async_copy(
                tT_hbm.at[:, pl.ds(m * 128, 128)], buf, sem
            )

        def out_desc(m, buf, sem):
            return pltpu.make_async_copy(
                buf, r_hbm.at[pl.ds(m * 64, D), :], sem
            )

        in_desc(m0, in0, si0).start()
        in_desc(m0 + 1, in1, si1).start()

        def body(u, carry):
            ma = m0 + 2 * u
            for (mb, in_v, out_v, si, so) in (
                (ma, in0, o0, si0, so0),
                (ma + 1, in1, o1, si1, so1),
            ):
                in_desc(mb, in_v, si).wait()
                pl.when(u > 0)(lambda: out_desc(mb - 2, out_v, so).wait())
                _repack_block(in_v, out_v, D)
                out_desc(mb, out_v, so).start()
                pl.when(u < RPW // 2 - 1)(
                    lambda: in_desc(mb + 2, in_v, si).start()
                )
            return carry

        lax.fori_loop(0, RPW // 2, body, 0)
        out_desc(m0 + RPW - 2, o0, so0).wait()
        out_desc(m0 + RPW - 1, o1, so1).wait()

        @pl.when(wid < 4)
        def _extra():
            m = NFULL - 4 + wid
            in_desc(m, in0, si0).start()
            in_desc(m, in0, si0).wait()
            _repack_block(in0, o0, D)
            out_desc(m, o0, so0).start()
            out_desc(m, o0, so0).wait()

        @pl.when(wid == NW - 1)
        def _tail():
            tin = pltpu.make_async_copy(
                tT_hbm.at[:, pl.ds(FULL_COLS, 64)], int_, si1
            )
            tin.start()
            tin.wait()
            _repack_block(int_, o1, 32)
            tout = pltpu.make_async_copy(
                o1.at[pl.ds(0, 32), :],
                r_hbm.at[pl.ds(FULL_COLS // 2, 32), :],
                so1,
            )
            tout.start()
            tout.wait()

    return repack


def _build_gather():
    mesh = plsc.VectorSubcoreMesh(core_axis_name="c", subcore_axis_name="s")

    @functools.partial(
        pl.kernel,
        mesh=mesh,
        out_type=jax.ShapeDtypeStruct((200, D, 4096), jnp.float32),
        scratch_types=[
            pltpu.VMEM((128,), jnp.int32),
            pltpu.VMEM((128,), jnp.int32),
            pltpu.VMEM((128,), jnp.int32),
            pltpu.VMEM((128,), jnp.int32),
            pltpu.VMEM((128,), jnp.int32),
            pltpu.VMEM((128,), jnp.int32),
            pltpu.VMEM((128, 128), jnp.float32),
            pltpu.VMEM((128, 128), jnp.float32),
            pltpu.VMEM((D, 128), jnp.float32),
            pltpu.VMEM((D, 128), jnp.float32),
            pltpu.SemaphoreType.DMA,
            pltpu.SemaphoreType.DMA,
            pltpu.SemaphoreType.DMA,
            pltpu.SemaphoreType.DMA,
            pltpu.SemaphoreType.DMA,
            pltpu.SemaphoreType.DMA,
        ],
        compiler_params=pltpu.CompilerParams(use_tc_tiling_on_sc=True, needs_layout_passes=False),
    )
    def gather(
        xf_hbm, r_hbm, out_hbm,
        ix0, ix1, aj0, aj1, hf0, hf1, rw0, rw1, sl0, sl1,
        sx0, sx1, sg0, sg1, so0, so1,
    ):
        wid = lax.axis_index("s") * NC + lax.axis_index("c")
        b0 = wid * BLK_PER_W

        def idx_desc(b, buf, sem):
            return pltpu.make_async_copy(
                xf_hbm.at[pl.ds(b * 128, 128)], buf, sem
            )

        def gat_desc(adj, buf, sem):
            return pltpu.make_async_copy(r_hbm.at[adj], buf, sem)

        def out_desc(b, buf, sem):
            j = lax.shift_right_logical(b, 5)
            i_hi = lax.bitwise_and(b, 31)
            return pltpu.make_async_copy(
                buf, out_hbm.at[j, :, pl.ds(i_hi * 128, 128)], sem
            )

        def arith(ix, adj, hf):
            for k in range(8):
                v = ix[pl.ds(16 * k, 16)]
                adj[pl.ds(16 * k, 16)] = lax.shift_right_logical(v, 1)
                hf[pl.ds(16 * k, 16)] = lax.bitwise_and(v, 1)

        def transpose(rw, hf, sl):
            i_vecs = [_iota16() + 16 * g for g in range(8)]
            hv64 = [hf[pl.ds(16 * g, 16)] * 64 for g in range(8)]

            @plsc.parallel_loop(0, D, unroll=8)
            def row(d):
                for g in range(8):
                    vals = plsc.load_gather(rw, [i_vecs[g], hv64[g] + d])
                    sl[d, pl.ds(16 * g, 16)] = vals

        idx_desc(b0, ix0, sx0).start()
        idx_desc(b0, ix0, sx0).wait()
        arith(ix0, aj0, hf0)
        gat_desc(aj0, rw0, sg0).start()
        idx_desc(b0 + 1, ix1, sx1).start()
        idx_desc(b0 + 1, ix1, sx1).wait()
        arith(ix1, aj1, hf1)
        gat_desc(aj1, rw1, sg1).start()

        def body(u, carry):
            ba = b0 + 2 * u
            for (bb, ix, aj, hf, rw, sl, sx, sg, so) in (
                (ba, ix0, aj0, hf0, rw0, sl0, sx0, sg0, so0),
                (ba + 1, ix1, aj1, hf1, rw1, sl1, sx1, sg1, so1),
            ):
                gat_desc(aj, rw, sg).wait()
                pl.when(u > 0)(lambda: out_desc(bb - 2, sl, so).wait())
                transpose(rw, hf, sl)
                out_desc(bb, sl, so).start()

                @pl.when(u < BLK_PER_W // 2 - 1)
                def _next():
                    idx_desc(bb + 2, ix, sx).start()
                    idx_desc(bb + 2, ix, sx).wait()
                    arith(ix, aj, hf)
                    gat_desc(aj, rw, sg).start()

            return carry

        lax.fori_loop(0, BLK_PER_W // 2, body, 0)
        out_desc(b0 + BLK_PER_W - 2, sl0, so0).wait()
        out_desc(b0 + BLK_PER_W - 1, sl1, so1).wait()

    return gather


def kernel(x, table):
    xf = x.T.reshape(B).astype(jnp.int32)
    tT = table.T
    repacked = _build_repack()(tT)
    out2 = _build_gather()(xf, repacked)
    return out2.transpose(2, 0, 1)


# speedup vs baseline: 3.1972x; 1.4099x over previous
import functools

import jax
import jax.numpy as jnp
from jax import lax
from jax.experimental import pallas as pl
from jax.experimental.pallas import tpu as pltpu
from jax.experimental.pallas import tpu_sc as plsc

V = 1_000_000
D = 64
VP = V // 2
NC, NS = 2, 16
NW = NC * NS
B = 4096 * 200
NBLK = B // 128
BLK_PER_W = NBLK // NW
FULL_COLS = (V // 128) * 128
NFULL = FULL_COLS // 128
RPW = NFULL // NW


def _iota16():
    return lax.iota(jnp.int32, 16)


def _repack_block(in_v, out_v, nrows):
    d_vecs = [_iota16() + 16 * q for q in range(4)]

    @plsc.parallel_loop(0, nrows, unroll=8)
    def row(r):
        for g in range(8):
            h = g // 4
            col = jnp.full((16,), 1, jnp.int32) * (2 * r + h)
            vals = plsc.load_gather(in_v, [d_vecs[g % 4], col])
            out_v[r, pl.ds(16 * g, 16)] = vals


def _build_repack():
    mesh = plsc.VectorSubcoreMesh(core_axis_name="c", subcore_axis_name="s")

    @functools.partial(
        pl.kernel,
        mesh=mesh,
        out_type=jax.ShapeDtypeStruct((VP, 128), jnp.float32),
        scratch_types=[
            pltpu.VMEM((D, 131), jnp.float32),
            pltpu.VMEM((D, 131), jnp.float32),
            pltpu.VMEM((D, 64), jnp.float32),
            pltpu.VMEM((D, 128), jnp.float32),
            pltpu.VMEM((D, 128), jnp.float32),
            pltpu.SemaphoreType.DMA,
            pltpu.SemaphoreType.DMA,
            pltpu.SemaphoreType.DMA,
            pltpu.SemaphoreType.DMA,
        ],
        compiler_params=pltpu.CompilerParams(use_tc_tiling_on_sc=True, needs_layout_passes=False),
    )
    def repack(tT_hbm, r_hbm, in0, in1, int_, o0, o1, si0, si1, so0, so1):
        wid = lax.axis_index("s") * NC + lax.axis_index("c")
        m0 = wid * RPW

        def in_desc(m, buf, sem):
            return pltpu.make_async_copy(
                tT_hbm.at[:, pl.ds(m * 128, 128)], buf.at[:, pl.ds(0, 128)], sem
            )

        def out_desc(m, buf, sem):
            return pltpu.make_async_copy(
                buf, r_hbm.at[pl.ds(m * 64, D), :], sem
            )

        in_desc(m0, in0, si0).start()
        in_desc(m0 + 1, in1, si1).start()

        def body(u, carry):
            ma = m0 + 2 * u
            for (mb, in_v, out_v, si, so) in (
                (ma, in0, o0, si0, so0),
                (ma + 1, in1, o1, si1, so1),
            ):
                in_desc(mb, in_v, si).wait()
                pl.when(u > 0)(lambda: out_desc(mb - 2, out_v, so).wait())
                _repack_block(in_v, out_v, D)
                out_desc(mb, out_v, so).start()
                pl.when(u < RPW // 2 - 1)(
                    lambda: in_desc(mb + 2, in_v, si).start()
                )
            return carry

        lax.fori_loop(0, RPW // 2, body, 0)
        out_desc(m0 + RPW - 2, o0, so0).wait()
        out_desc(m0 + RPW - 1, o1, so1).wait()

        @pl.when(wid < 4)
        def _extra():
            m = NFULL - 4 + wid
            in_desc(m, in0, si0).start()
            in_desc(m, in0, si0).wait()
            _repack_block(in0, o0, D)
            out_desc(m, o0, so0).start()
            out_desc(m, o0, so0).wait()

        @pl.when(wid == NW - 1)
        def _tail():
            tin = pltpu.make_async_copy(
                tT_hbm.at[:, pl.ds(FULL_COLS, 64)], int_, si1
            )
            tin.start()
            tin.wait()
            _repack_block(int_, o1, 32)
            tout = pltpu.make_async_copy(
                o1.at[pl.ds(0, 32), :],
                r_hbm.at[pl.ds(FULL_COLS // 2, 32), :],
                so1,
            )
            tout.start()
            tout.wait()

    return repack


def _build_gather():
    mesh = plsc.VectorSubcoreMesh(core_axis_name="c", subcore_axis_name="s")

    @functools.partial(
        pl.kernel,
        mesh=mesh,
        out_type=jax.ShapeDtypeStruct((200, D, 4096), jnp.float32),
        scratch_types=[
            pltpu.VMEM((128,), jnp.int32),
            pltpu.VMEM((128,), jnp.int32),
            pltpu.VMEM((128,), jnp.int32),
            pltpu.VMEM((128,), jnp.int32),
            pltpu.VMEM((128,), jnp.int32),
            pltpu.VMEM((128,), jnp.int32),
            pltpu.VMEM((128, 128), jnp.float32),
            pltpu.VMEM((128, 128), jnp.float32),
            pltpu.VMEM((D, 128), jnp.float32),
            pltpu.VMEM((D, 128), jnp.float32),
            pltpu.SemaphoreType.DMA,
            pltpu.SemaphoreType.DMA,
            pltpu.SemaphoreType.DMA,
            pltpu.SemaphoreType.DMA,
            pltpu.SemaphoreType.DMA,
            pltpu.SemaphoreType.DMA,
        ],
        compiler_params=pltpu.CompilerParams(use_tc_tiling_on_sc=True, needs_layout_passes=False),
    )
    def gather(
        xf_hbm, r_hbm, out_hbm,
        ix0, ix1, aj0, aj1, hf0, hf1, rw0, rw1, sl0, sl1,
        sx0, sx1, sg0, sg1, so0, so1,
    ):
        wid = lax.axis_index("s") * NC + lax.axis_index("c")
        b0 = wid * BLK_PER_W

        def idx_desc(b, buf, sem):
            return pltpu.make_async_copy(
                xf_hbm.at[pl.ds(b * 128, 128)], buf, sem
            )

        def gat_desc(adj, buf, sem):
            return pltpu.make_async_copy(r_hbm.at[adj], buf, sem)

        def out_desc(b, buf, sem):
            j = lax.shift_right_logical(b, 5)
            i_hi = lax.bitwise_and(b, 31)
            return pltpu.make_async_copy(
                buf, out_hbm.at[j, :, pl.ds(i_hi * 128, 128)], sem
            )

        def arith(ix, adj, hf):
            for k in range(8):
                v = ix[pl.ds(16 * k, 16)]
                adj[pl.ds(16 * k, 16)] = lax.shift_right_logical(v, 1)
                hf[pl.ds(16 * k, 16)] = lax.bitwise_and(v, 1)

        def transpose(rw, hf, sl):
            i_vecs = [_iota16() + 16 * g for g in range(8)]
            hv64 = [hf[pl.ds(16 * g, 16)] * 64 for g in range(8)]
            rot = _iota16()

            @plsc.parallel_loop(0, D, unroll=8)
            def row(d):
                d_vec = lax.bitwise_and(rot + d, 63)
                for g in range(8):
                    vals = plsc.load_gather(rw, [i_vecs[g], hv64[g] + d_vec])
                    plsc.store_scatter(sl, [d_vec, i_vecs[g]], vals)

        idx_desc(b0, ix0, sx0).start()
        idx_desc(b0, ix0, sx0).wait()
        arith(ix0, aj0, hf0)
        gat_desc(aj0, rw0, sg0).start()
        idx_desc(b0 + 1, ix1, sx1).start()
        idx_desc(b0 + 1, ix1, sx1).wait()
        arith(ix1, aj1, hf1)
        gat_desc(aj1, rw1, sg1).start()

        def body(u, carry):
            ba = b0 + 2 * u
            for (bb, ix, aj, hf, rw, sl, sx, sg, so) in (
                (ba, ix0, aj0, hf0, rw0, sl0, sx0, sg0, so0),
                (ba + 1, ix1, aj1, hf1, rw1, sl1, sx1, sg1, so1),
            ):
                gat_desc(aj, rw, sg).wait()
                pl.when(u > 0)(lambda: out_desc(bb - 2, sl, so).wait())
                transpose(rw, hf, sl)
                out_desc(bb, sl, so).start()

                @pl.when(u < BLK_PER_W // 2 - 1)
                def _next():
                    idx_desc(bb + 2, ix, sx).start()
                    idx_desc(bb + 2, ix, sx).wait()
                    arith(ix, aj, hf)
                    gat_desc(aj, rw, sg).start()

            return carry

        lax.fori_loop(0, BLK_PER_W // 2, body, 0)
        out_desc(b0 + BLK_PER_W - 2, sl0, so0).wait()
        out_desc(b0 + BLK_PER_W - 1, sl1, so1).wait()

    return gather


def kernel(x, table):
    xf = x.T.reshape(B).astype(jnp.int32)
    tT = table.T
    repacked = _build_repack()(tT)
    out2 = _build_gather()(xf, repacked)
    return out2.transpose(2, 0, 1)


# speedup vs baseline: 6.4053x; 2.0034x over previous
import functools

import jax
import jax.numpy as jnp
from jax import lax
from jax.experimental import pallas as pl
from jax.experimental.pallas import tpu as pltpu
from jax.experimental.pallas import tpu_sc as plsc

V = 1_000_000
D = 64
VP = V // 2
NC, NS = 2, 16
NW = NC * NS
B = 4096 * 200
NBLK = B // 128
BLK_PER_W = NBLK // NW
FULL_COLS = (V // 128) * 128
NFULL = FULL_COLS // 128
RPW = NFULL // NW


def _iota16():
    return lax.iota(jnp.int32, 16)


def _repack_block(in_v, out_v, nrows):
    rot = _iota16()
    r_vec = lax.shift_right_logical(rot, 1)
    h64_vec = lax.bitwise_and(rot, 1) * 64

    for q in range(nrows // 8):
        col_vec = rot + 16 * q
        r_q = r_vec + 8 * q

        @plsc.parallel_loop(0, D, unroll=8)
        def dim(d):
            d_vec = lax.bitwise_and(rot + d, 63)
            vals = plsc.load_gather(in_v, [d_vec, col_vec])
            plsc.store_scatter(out_v, [r_q, d_vec + h64_vec], vals)


def _build_repack():
    mesh = plsc.VectorSubcoreMesh(core_axis_name="c", subcore_axis_name="s")

    @functools.partial(
        pl.kernel,
        mesh=mesh,
        out_type=jax.ShapeDtypeStruct((VP, 128), jnp.float32),
        scratch_types=[
            pltpu.VMEM((D, 128), jnp.float32),
            pltpu.VMEM((D, 128), jnp.float32),
            pltpu.VMEM((D, 64), jnp.float32),
            pltpu.VMEM((D, 128), jnp.float32),
            pltpu.VMEM((D, 128), jnp.float32),
            pltpu.SemaphoreType.DMA,
            pltpu.SemaphoreType.DMA,
            pltpu.SemaphoreType.DMA,
            pltpu.SemaphoreType.DMA,
        ],
        compiler_params=pltpu.CompilerParams(use_tc_tiling_on_sc=True, needs_layout_passes=False),
    )
    def repack(tT_hbm, r_hbm, in0, in1, int_, o0, o1, si0, si1, so0, so1):
        wid = lax.axis_index("s") * NC + lax.axis_index("c")
        m0 = wid * RPW

        def in_desc(m, buf, sem):
            return pltpu.make_async_copy(
                tT_hbm.at[:, pl.ds(m * 128, 128)], buf, sem
            )

        def out_desc(m, buf, sem):
            return pltpu.make_async_copy(
                buf, r_hbm.at[pl.ds(m * 64, D), :], sem
            )

        in_desc(m0, in0, si0).start()
        in_desc(m0 + 1, in1, si1).start()

        def body(u, carry):
            ma = m0 + 2 * u
            for (mb, in_v, out_v, si, so) in (
                (ma, in0, o0, si0, so0),
                (ma + 1, in1, o1, si1, so1),
            ):
                in_desc(mb, in_v, si).wait()
                pl.when(u > 0)(lambda: out_desc(mb - 2, out_v, so).wait())
                _repack_block(in_v, out_v, D)
                out_desc(mb, out_v, so).start()
                pl.when(u < RPW // 2 - 1)(
                    lambda: in_desc(mb + 2, in_v, si).start()
                )
            return carry

        lax.fori_loop(0, RPW // 2, body, 0)
        out_desc(m0 + RPW - 2, o0, so0).wait()
        out_desc(m0 + RPW - 1, o1, so1).wait()

        @pl.when(wid < 4)
        def _extra():
            m = NFULL - 4 + wid
            in_desc(m, in0, si0).start()
            in_desc(m, in0, si0).wait()
            _repack_block(in0, o0, D)
            out_desc(m, o0, so0).start()
            out_desc(m, o0, so0).wait()

        @pl.when(wid == NW - 1)
        def _tail():
            tin = pltpu.make_async_copy(
                tT_hbm.at[:, pl.ds(FULL_COLS, 64)], int_, si1
            )
            tin.start()
            tin.wait()
            _repack_block(int_, o1, 32)
            tout = pltpu.make_async_copy(
                o1.at[pl.ds(0, 32), :],
                r_hbm.at[pl.ds(FULL_COLS // 2, 32), :],
                so1,
            )
            tout.start()
            tout.wait()

    return repack


def _build_gather():
    mesh = plsc.VectorSubcoreMesh(core_axis_name="c", subcore_axis_name="s")

    @functools.partial(
        pl.kernel,
        mesh=mesh,
        out_type=jax.ShapeDtypeStruct((200, D, 4096), jnp.float32),
        scratch_types=[
            pltpu.VMEM((128,), jnp.int32),
            pltpu.VMEM((128,), jnp.int32),
            pltpu.VMEM((128,), jnp.int32),
            pltpu.VMEM((128,), jnp.int32),
            pltpu.VMEM((128,), jnp.int32),
            pltpu.VMEM((128,), jnp.int32),
            pltpu.VMEM((128, 128), jnp.float32),
            pltpu.VMEM((128, 128), jnp.float32),
            pltpu.VMEM((D, 128), jnp.float32),
            pltpu.VMEM((D, 128), jnp.float32),
            pltpu.SemaphoreType.DMA,
            pltpu.SemaphoreType.DMA,
            pltpu.SemaphoreType.DMA,
            pltpu.SemaphoreType.DMA,
            pltpu.SemaphoreType.DMA,
            pltpu.SemaphoreType.DMA,
        ],
        compiler_params=pltpu.CompilerParams(use_tc_tiling_on_sc=True, needs_layout_passes=False),
    )
    def gather(
        xf_hbm, r_hbm, out_hbm,
        ix0, ix1, aj0, aj1, hf0, hf1, rw0, rw1, sl0, sl1,
        sx0, sx1, sg0, sg1, so0, so1,
    ):
        wid = lax.axis_index("s") * NC + lax.axis_index("c")
        b0 = wid * BLK_PER_W

        def idx_desc(b, buf, sem):
            return pltpu.make_async_copy(
                xf_hbm.at[pl.ds(b * 128, 128)], buf, sem
            )

        def gat_desc(adj, buf, sem):
            return pltpu.make_async_copy(r_hbm.at[adj], buf, sem)

        def out_desc(b, buf, sem):
            j = lax.shift_right_logical(b, 5)
            i_hi = lax.bitwise_and(b, 31)
            return pltpu.make_async_copy(
                buf, out_hbm.at[j, :, pl.ds(i_hi * 128, 128)], sem
            )

        def arith(ix, adj, hf):
            for k in range(8):
                v = ix[pl.ds(16 * k, 16)]
                adj[pl.ds(16 * k, 16)] = lax.shift_right_logical(v, 1)
                hf[pl.ds(16 * k, 16)] = lax.bitwise_and(v, 1)

        def transpose(rw, hf, sl):
            i_vecs = [_iota16() + 16 * g for g in range(8)]
            hv64 = [hf[pl.ds(16 * g, 16)] * 64 for g in range(8)]
            rot = _iota16()

            @plsc.parallel_loop(0, D, unroll=8)
            def row(d):
                d_vec = lax.bitwise_and(rot + d, 63)
                for g in range(8):
                    vals = plsc.load_gather(rw, [i_vecs[g], hv64[g] + d_vec])
                    plsc.store_scatter(sl, [d_vec, i_vecs[g]], vals)

        idx_desc(b0, ix0, sx0).start()
        idx_desc(b0, ix0, sx0).wait()
        arith(ix0, aj0, hf0)
        gat_desc(aj0, rw0, sg0).start()
        idx_desc(b0 + 1, ix1, sx1).start()
        idx_desc(b0 + 1, ix1, sx1).wait()
        arith(ix1, aj1, hf1)
        gat_desc(aj1, rw1, sg1).start()

        def body(u, carry):
            ba = b0 + 2 * u
            for (bb, ix, aj, hf, rw, sl, sx, sg, so) in (
                (ba, ix0, aj0, hf0, rw0, sl0, sx0, sg0, so0),
                (ba + 1, ix1, aj1, hf1, rw1, sl1, sx1, sg1, so1),
            ):
                gat_desc(aj, rw, sg).wait()
                pl.when(u > 0)(lambda: out_desc(bb - 2, sl, so).wait())
                transpose(rw, hf, sl)
                out_desc(bb, sl, so).start()

                @pl.when(u < BLK_PER_W // 2 - 1)
                def _next():
                    idx_desc(bb + 2, ix, sx).start()
                    idx_desc(bb + 2, ix, sx).wait()
                    arith(ix, aj, hf)
                    gat_desc(aj, rw, sg).start()

            return carry

        lax.fori_loop(0, BLK_PER_W // 2, body, 0)
        out_desc(b0 + BLK_PER_W - 2, sl0, so0).wait()
        out_desc(b0 + BLK_PER_W - 1, sl1, so1).wait()

    return gather


def kernel(x, table):
    xf = x.T.reshape(B).astype(jnp.int32)
    tT = table.T
    repacked = _build_repack()(tT)
    out2 = _build_gather()(xf, repacked)
    return out2.transpose(2, 0, 1)


# speedup vs baseline: 7.1561x; 1.1172x over previous
import functools

import jax
import jax.numpy as jnp
from jax import lax
from jax.experimental import pallas as pl
from jax.experimental.pallas import tpu as pltpu
from jax.experimental.pallas import tpu_sc as plsc

V = 1_000_000
D = 64
VP = V // 2
NC, NS = 2, 16
NW = NC * NS
B = 4096 * 200
NBLK = B // 128
BLK_PER_W = NBLK // NW
FULL_COLS = (V // 128) * 128
NFULL = FULL_COLS // 128
RPW = NFULL // NW


def _iota16():
    return lax.iota(jnp.int32, 16)


def _repack_block(in_v, out_v, nrows):
    rot = _iota16()
    r_vec = lax.shift_right_logical(rot, 1)
    h64_vec = lax.bitwise_and(rot, 1) * 64

    for q in range(nrows // 8):
        col_vec = rot + 16 * q
        r_q = r_vec + 8 * q

        @plsc.parallel_loop(0, D, unroll=8)
        def dim(d):
            d_vec = lax.bitwise_and(rot + d, 63)
            vals = plsc.load_gather(in_v, [d_vec, col_vec])
            plsc.store_scatter(out_v, [r_q, d_vec + h64_vec], vals)


def _build_repack():
    mesh = plsc.VectorSubcoreMesh(core_axis_name="c", subcore_axis_name="s")

    @functools.partial(
        pl.kernel,
        mesh=mesh,
        out_type=jax.ShapeDtypeStruct((VP, 128), jnp.float32),
        scratch_types=[
            pltpu.VMEM((D, 128), jnp.float32),
            pltpu.VMEM((D, 128), jnp.float32),
            pltpu.VMEM((D, 64), jnp.float32),
            pltpu.VMEM((D, 128), jnp.float32),
            pltpu.VMEM((D, 128), jnp.float32),
            pltpu.SemaphoreType.DMA,
            pltpu.SemaphoreType.DMA,
            pltpu.SemaphoreType.DMA,
            pltpu.SemaphoreType.DMA,
        ],
        compiler_params=pltpu.CompilerParams(use_tc_tiling_on_sc=True, needs_layout_passes=False),
    )
    def repack(tT_hbm, r_hbm, in0, in1, int_, o0, o1, si0, si1, so0, so1):
        wid = lax.axis_index("s") * NC + lax.axis_index("c")
        m0 = wid * RPW

        def in_desc(m, buf, sem):
            return pltpu.make_async_copy(
                tT_hbm.at[:, pl.ds(m * 128, 128)], buf, sem
            )

        def out_desc(m, buf, sem):
            return pltpu.make_async_copy(
                buf, r_hbm.at[pl.ds(m * 64, D), :], sem
            )

        in_desc(m0, in0, si0).start()
        in_desc(m0 + 1, in1, si1).start()

        def body(u, carry):
            ma = m0 + 2 * u
            for (mb, in_v, out_v, si, so) in (
                (ma, in0, o0, si0, so0),
                (ma + 1, in1, o1, si1, so1),
            ):
                in_desc(mb, in_v, si).wait()
                pl.when(u > 0)(lambda: out_desc(mb - 2, out_v, so).wait())
                _repack_block(in_v, out_v, D)
                out_desc(mb, out_v, so).start()
                pl.when(u < RPW // 2 - 1)(
                    lambda: in_desc(mb + 2, in_v, si).start()
                )
            return carry

        lax.fori_loop(0, RPW // 2, body, 0)
        out_desc(m0 + RPW - 2, o0, so0).wait()
        out_desc(m0 + RPW - 1, o1, so1).wait()

        @pl.when(wid < 4)
        def _extra():
            m = NFULL - 4 + wid
            in_desc(m, in0, si0).start()
            in_desc(m, in0, si0).wait()
            _repack_block(in0, o0, D)
            out_desc(m, o0, so0).start()
            out_desc(m, o0, so0).wait()

        @pl.when(wid == NW - 1)
        def _tail():
            tin = pltpu.make_async_copy(
                tT_hbm.at[:, pl.ds(FULL_COLS, 64)], int_, si1
            )
            tin.start()
            tin.wait()
            _repack_block(int_, o1, 32)
            tout = pltpu.make_async_copy(
                o1.at[pl.ds(0, 32), :],
                r_hbm.at[pl.ds(FULL_COLS // 2, 32), :],
                so1,
            )
            tout.start()
            tout.wait()

    return repack


def _build_gather():
    mesh = plsc.VectorSubcoreMesh(core_axis_name="c", subcore_axis_name="s")

    @functools.partial(
        pl.kernel,
        mesh=mesh,
        out_type=jax.ShapeDtypeStruct((200, 8, 32, 8, 128), jnp.float32),
        scratch_types=[
            pltpu.VMEM((128,), jnp.int32),
            pltpu.VMEM((128,), jnp.int32),
            pltpu.VMEM((128, D), jnp.float32),
            pltpu.VMEM((128, D), jnp.float32),
            pltpu.VMEM((8, 8, 128), jnp.float32),
            pltpu.VMEM((8, 8, 128), jnp.float32),
            pltpu.SemaphoreType.DMA,
            pltpu.SemaphoreType.DMA,
            pltpu.SemaphoreType.DMA,
            pltpu.SemaphoreType.DMA,
            pltpu.SemaphoreType.DMA,
            pltpu.SemaphoreType.DMA,
        ],
        compiler_params=pltpu.CompilerParams(use_tc_tiling_on_sc=False, needs_layout_passes=False),
    )
    def gather(
        xf_hbm, r_hbm, out_hbm,
        ix0, ix1, rw0, rw1, sl0, sl1,
        sx0, sx1, sg0, sg1, so0, so1,
    ):
        wid = lax.axis_index("s") * NC + lax.axis_index("c")
        b0 = wid * BLK_PER_W

        def idx_desc(b, buf, sem):
            return pltpu.make_async_copy(
                xf_hbm.at[pl.ds(b * 128, 128)], buf, sem
            )

        def gat_desc(ix, buf, sem):
            return pltpu.make_async_copy(r_hbm.at[ix], buf, sem)

        def out_desc(b, buf, sem):
            j = lax.shift_right_logical(b, 5)
            i_hi = lax.bitwise_and(b, 31)
            return pltpu.make_async_copy(
                buf, out_hbm.at[j, :, i_hi, :, :], sem
            )

        def transpose(rw, sl):
            i_vecs = [_iota16() + 16 * g for g in range(8)]
            rot = _iota16()

            @plsc.parallel_loop(0, D, unroll=8)
            def row(d):
                d_vec = lax.bitwise_and(rot + d, 63)
                d_hi = lax.shift_right_logical(d_vec, 3)
                d_lo = lax.bitwise_and(d_vec, 7)
                for g in range(8):
                    vals = plsc.load_gather(rw, [i_vecs[g], d_vec])
                    plsc.store_scatter(sl, [d_hi, d_lo, i_vecs[g]], vals)

        idx_desc(b0, ix0, sx0).start()
        idx_desc(b0, ix0, sx0).wait()
        gat_desc(ix0, rw0, sg0).start()
        idx_desc(b0 + 1, ix1, sx1).start()
        idx_desc(b0 + 1, ix1, sx1).wait()
        gat_desc(ix1, rw1, sg1).start()

        def body(u, carry):
            ba = b0 + 2 * u
            for (bb, ix, rw, sl, sx, sg, so) in (
                (ba, ix0, rw0, sl0, sx0, sg0, so0),
                (ba + 1, ix1, rw1, sl1, sx1, sg1, so1),
            ):
                gat_desc(ix, rw, sg).wait()
                pl.when(u > 0)(lambda: out_desc(bb - 2, sl, so).wait())
                transpose(rw, sl)
                out_desc(bb, sl, so).start()

                @pl.when(u < BLK_PER_W // 2 - 1)
                def _next():
                    idx_desc(bb + 2, ix, sx).start()
                    idx_desc(bb + 2, ix, sx).wait()
                    gat_desc(ix, rw, sg).start()

            return carry

        lax.fori_loop(0, BLK_PER_W // 2, body, 0)
        out_desc(b0 + BLK_PER_W - 2, sl0, so0).wait()
        out_desc(b0 + BLK_PER_W - 1, sl1, so1).wait()

    return gather


def kernel(x, table):
    xf = x.T.reshape(B).astype(jnp.int32)
    tT = table.T
    repacked = _build_repack()(tT)
    r_lin = repacked.reshape(V, D)
    out5 = _build_gather()(xf, r_lin)
    return out5.transpose(2, 4, 0, 1, 3).reshape(4096, 200, D)


# speedup vs baseline: 8.0340x; 1.1227x over previous
import functools

import jax
import jax.numpy as jnp
from jax import lax
from jax.experimental import pallas as pl
from jax.experimental.pallas import tpu as pltpu
from jax.experimental.pallas import tpu_sc as plsc

V = 1_000_000
D = 64
VP = V // 2
NC, NS = 2, 16
NW = NC * NS
B = 4096 * 200
NBLK = B // 128
BLK_PER_W = NBLK // NW
FULL_COLS = (V // 128) * 128
NFULL = FULL_COLS // 128
RPW = NFULL // NW


def _iota16():
    return lax.iota(jnp.int32, 16)


def _repack_block(in_v, out_v, nrows):
    rot = _iota16()
    r_vec = lax.shift_right_logical(rot, 1)
    h64_vec = lax.bitwise_and(rot, 1) * 64

    for q in range(nrows // 8):
        col_vec = rot + 16 * q
        r_q = r_vec + 8 * q

        @plsc.parallel_loop(0, D, unroll=8)
        def dim(d):
            d_vec = lax.bitwise_and(rot + d, 63)
            vals = plsc.load_gather(in_v, [d_vec, col_vec])
            plsc.store_scatter(out_v, [r_q, d_vec + h64_vec], vals)


def _build_repack():
    mesh = plsc.VectorSubcoreMesh(core_axis_name="c", subcore_axis_name="s")

    @functools.partial(
        pl.kernel,
        mesh=mesh,
        out_type=jax.ShapeDtypeStruct((VP, 128), jnp.float32),
        scratch_types=[
            pltpu.VMEM((D, 128), jnp.float32),
            pltpu.VMEM((D, 128), jnp.float32),
            pltpu.VMEM((D, 64), jnp.float32),
            pltpu.VMEM((D, 128), jnp.float32),
            pltpu.VMEM((D, 128), jnp.float32),
            pltpu.SemaphoreType.DMA,
            pltpu.SemaphoreType.DMA,
            pltpu.SemaphoreType.DMA,
            pltpu.SemaphoreType.DMA,
        ],
        compiler_params=pltpu.CompilerParams(use_tc_tiling_on_sc=True, needs_layout_passes=False),
    )
    def repack(tT_hbm, r_hbm, in0, in1, int_, o0, o1, si0, si1, so0, so1):
        wid = lax.axis_index("s") * NC + lax.axis_index("c")
        m0 = wid * RPW

        def in_desc(m, buf, sem):
            return pltpu.make_async_copy(
                tT_hbm.at[:, pl.ds(m * 128, 128)], buf, sem
            )

        def out_desc(m, buf, sem):
            return pltpu.make_async_copy(
                buf, r_hbm.at[pl.ds(m * 64, D), :], sem
            )

        in_desc(m0, in0, si0).start()
        in_desc(m0 + 1, in1, si1).start()

        def body(u, carry):
            ma = m0 + 2 * u
            for (mb, in_v, out_v, si, so) in (
                (ma, in0, o0, si0, so0),
                (ma + 1, in1, o1, si1, so1),
            ):
                in_desc(mb, in_v, si).wait()
                pl.when(u > 0)(lambda: out_desc(mb - 2, out_v, so).wait())
                _repack_block(in_v, out_v, D)
                out_desc(mb, out_v, so).start()
                pl.when(u < RPW // 2 - 1)(
                    lambda: in_desc(mb + 2, in_v, si).start()
                )
            return carry

        lax.fori_loop(0, RPW // 2, body, 0)
        out_desc(m0 + RPW - 2, o0, so0).wait()
        out_desc(m0 + RPW - 1, o1, so1).wait()

        @pl.when(wid < 4)
        def _extra():
            m = NFULL - 4 + wid
            in_desc(m, in0, si0).start()
            in_desc(m, in0, si0).wait()
            _repack_block(in0, o0, D)
            out_desc(m, o0, so0).start()
            out_desc(m, o0, so0).wait()

        @pl.when(wid == NW - 1)
        def _tail():
            tin = pltpu.make_async_copy(
                tT_hbm.at[:, pl.ds(FULL_COLS, 64)], int_, si1
            )
            tin.start()
            tin.wait()
            _repack_block(int_, o1, 32)
            tout = pltpu.make_async_copy(
                o1.at[pl.ds(0, 32), :],
                r_hbm.at[pl.ds(FULL_COLS // 2, 32), :],
                so1,
            )
            tout.start()
            tout.wait()

    return repack


def _build_gather():
    mesh = plsc.VectorSubcoreMesh(core_axis_name="c", subcore_axis_name="s")

    @functools.partial(
        pl.kernel,
        mesh=mesh,
        out_type=jax.ShapeDtypeStruct((200, 8, 32, 8, 128), jnp.float32),
        scratch_types=[
            pltpu.VMEM((BLK_PER_W * 128,), jnp.int32),
            pltpu.VMEM((128, D), jnp.float32),
            pltpu.VMEM((128, D), jnp.float32),
            pltpu.VMEM((8, 8, 128), jnp.float32),
            pltpu.VMEM((8, 8, 128), jnp.float32),
            pltpu.SemaphoreType.DMA,
            pltpu.SemaphoreType.DMA,
            pltpu.SemaphoreType.DMA,
            pltpu.SemaphoreType.DMA,
            pltpu.SemaphoreType.DMA,
        ],
        compiler_params=pltpu.CompilerParams(use_tc_tiling_on_sc=False, needs_layout_passes=False),
    )
    def gather(
        xf_hbm, r_hbm, out_hbm,
        ixall, rw0, rw1, sl0, sl1,
        sxa, sg0, sg1, so0, so1,
    ):
        wid = lax.axis_index("s") * NC + lax.axis_index("c")
        b0 = wid * BLK_PER_W

        def gat_desc(t, buf, sem):
            return pltpu.make_async_copy(
                r_hbm.at[ixall.at[pl.ds(t * 128, 128)]], buf, sem
            )

        def out_desc(b, buf, sem):
            j = lax.shift_right_logical(b, 5)
            i_hi = lax.bitwise_and(b, 31)
            return pltpu.make_async_copy(
                buf, out_hbm.at[j, :, i_hi, :, :], sem
            )

        def transpose(rw, sl):
            i_vecs = [_iota16() + 16 * g for g in range(8)]
            rot = _iota16()

            @plsc.parallel_loop(0, D, unroll=8)
            def row(d):
                d_vec = lax.bitwise_and(rot + d, 63)
                d_hi = lax.shift_right_logical(d_vec, 3)
                d_lo = lax.bitwise_and(d_vec, 7)
                for g in range(8):
                    vals = plsc.load_gather(rw, [i_vecs[g], d_vec])
                    plsc.store_scatter(sl, [d_hi, d_lo, i_vecs[g]], vals)

        ixa_desc = pltpu.make_async_copy(
            xf_hbm.at[pl.ds(b0 * 128, BLK_PER_W * 128)], ixall, sxa
        )
        ixa_desc.start()
        ixa_desc.wait()
        gat_desc(0, rw0, sg0).start()
        gat_desc(1, rw1, sg1).start()

        def body(u, carry):
            for (k, rw, sl, sg, so) in (
                (0, rw0, sl0, sg0, so0),
                (1, rw1, sl1, sg1, so1),
            ):
                t = 2 * u + k
                bb = b0 + t
                gat_desc(t, rw, sg).wait()
                pl.when(u > 0)(lambda: out_desc(bb - 2, sl, so).wait())
                transpose(rw, sl)
                out_desc(bb, sl, so).start()
                pl.when(u < BLK_PER_W // 2 - 1)(
                    lambda: gat_desc(t + 2, rw, sg).start()
                )

            return carry

        lax.fori_loop(0, BLK_PER_W // 2, body, 0)
        out_desc(b0 + BLK_PER_W - 2, sl0, so0).wait()
        out_desc(b0 + BLK_PER_W - 1, sl1, so1).wait()

    return gather


def kernel(x, table):
    xf = x.T.reshape(B).astype(jnp.int32)
    tT = table.T
    repacked = _build_repack()(tT)
    r_lin = repacked.reshape(V, D)
    out5 = _build_gather()(xf, r_lin)
    return out5.transpose(2, 4, 0, 1, 3).reshape(4096, 200, D)


# speedup vs baseline: 8.5463x; 1.0638x over previous
import functools

import jax
import jax.numpy as jnp
from jax import lax
from jax.experimental import pallas as pl
from jax.experimental.pallas import tpu as pltpu
from jax.experimental.pallas import tpu_sc as plsc

V = 1_000_000
D = 64
VP = V // 2
NC, NS = 2, 16
NW = NC * NS
B = 4096 * 200
NBLK = B // 128
BLK_PER_W = NBLK // NW
FULL_COLS = (V // 128) * 128
NFULL2 = FULL_COLS // 256
RPW = (NFULL2 // NW) & ~1
NEXTRA = NFULL2 - RPW * NW


def _iota16():
    return lax.iota(jnp.int32, 16)


def _repack_block(in_v, out_v, nrows):
    rot = _iota16()
    r_vec = lax.shift_right_logical(rot, 1)
    h64_vec = lax.bitwise_and(rot, 1) * 64

    for q in range(nrows // 8):
        col_vec = rot + 16 * q
        r_q = r_vec + 8 * q

        @plsc.parallel_loop(0, D, unroll=8)
        def dim(d):
            d_vec = lax.bitwise_and(rot + d, 63)
            vals = plsc.load_gather(in_v, [d_vec, col_vec])
            plsc.store_scatter(out_v, [r_q, d_vec + h64_vec], vals)


def _build_repack():
    mesh = plsc.VectorSubcoreMesh(core_axis_name="c", subcore_axis_name="s")

    @functools.partial(
        pl.kernel,
        mesh=mesh,
        out_type=jax.ShapeDtypeStruct((VP, 128), jnp.float32),
        scratch_types=[
            pltpu.VMEM((D, 256), jnp.float32),
            pltpu.VMEM((D, 256), jnp.float32),
            pltpu.VMEM((D, 64), jnp.float32),
            pltpu.VMEM((128, 128), jnp.float32),
            pltpu.VMEM((128, 128), jnp.float32),
            pltpu.SemaphoreType.DMA,
            pltpu.SemaphoreType.DMA,
            pltpu.SemaphoreType.DMA,
            pltpu.SemaphoreType.DMA,
        ],
        compiler_params=pltpu.CompilerParams(use_tc_tiling_on_sc=True, needs_layout_passes=False),
    )
    def repack(tT_hbm, r_hbm, in0, in1, int_, o0, o1, si0, si1, so0, so1):
        wid = lax.axis_index("s") * NC + lax.axis_index("c")
        m0 = wid * RPW

        def in_desc(m, buf, sem):
            return pltpu.make_async_copy(
                tT_hbm.at[:, pl.ds(m * 256, 256)], buf, sem
            )

        def out_desc(m, buf, sem):
            return pltpu.make_async_copy(
                buf, r_hbm.at[pl.ds(m * 128, 128), :], sem
            )

        in_desc(m0, in0, si0).start()
        in_desc(m0 + 1, in1, si1).start()

        def body(u, carry):
            ma = m0 + 2 * u
            for (mb, in_v, out_v, si, so) in (
                (ma, in0, o0, si0, so0),
                (ma + 1, in1, o1, si1, so1),
            ):
                in_desc(mb, in_v, si).wait()
                pl.when(u > 0)(lambda: out_desc(mb - 2, out_v, so).wait())
                _repack_block(in_v, out_v, 128)
                out_desc(mb, out_v, so).start()
                pl.when(u < RPW // 2 - 1)(
                    lambda: in_desc(mb + 2, in_v, si).start()
                )
            return carry

        lax.fori_loop(0, RPW // 2, body, 0)
        out_desc(m0 + RPW - 2, o0, so0).wait()
        out_desc(m0 + RPW - 1, o1, so1).wait()

        @pl.when(wid < NEXTRA)
        def _extra():
            m = NFULL2 - NEXTRA + wid
            in_desc(m, in0, si0).start()
            in_desc(m, in0, si0).wait()
            _repack_block(in0, o0, 128)
            out_desc(m, o0, so0).start()
            out_desc(m, o0, so0).wait()

        @pl.when(wid == NW - 1)
        def _tail():
            tin = pltpu.make_async_copy(
                tT_hbm.at[:, pl.ds(FULL_COLS, 64)], int_, si1
            )
            tin.start()
            tin.wait()
            _repack_block(int_, o1, 32)
            tout = pltpu.make_async_copy(
                o1.at[pl.ds(0, 32), :],
                r_hbm.at[pl.ds(FULL_COLS // 2, 32), :],
                so1,
            )
            tout.start()
            tout.wait()

    return repack


def _build_gather():
    mesh = plsc.VectorSubcoreMesh(core_axis_name="c", subcore_axis_name="s")

    @functools.partial(
        pl.kernel,
        mesh=mesh,
        out_type=jax.ShapeDtypeStruct((200, 8, 32, 8, 128), jnp.float32),
        scratch_types=[
            pltpu.VMEM((BLK_PER_W * 128,), jnp.int32),
            pltpu.VMEM((128, D), jnp.float32),
            pltpu.VMEM((128, D), jnp.float32),
            pltpu.VMEM((128, D), jnp.float32),
            pltpu.VMEM((128, D), jnp.float32),
            pltpu.VMEM((8, 8, 128), jnp.float32),
            pltpu.VMEM((8, 8, 128), jnp.float32),
            pltpu.VMEM((8, 8, 128), jnp.float32),
            pltpu.VMEM((8, 8, 128), jnp.float32),
            pltpu.SemaphoreType.DMA,
            pltpu.SemaphoreType.DMA,
            pltpu.SemaphoreType.DMA,
            pltpu.SemaphoreType.DMA,
            pltpu.SemaphoreType.DMA,
            pltpu.SemaphoreType.DMA,
            pltpu.SemaphoreType.DMA,
            pltpu.SemaphoreType.DMA,
            pltpu.SemaphoreType.DMA,
        ],
        compiler_params=pltpu.CompilerParams(use_tc_tiling_on_sc=False, needs_layout_passes=False),
    )
    def gather(
        xf_hbm, r_hbm, out_hbm,
        ixall, rw0, rw1, rw2, rw3, sl0, sl1, sl2, sl3,
        sxa, sg0, sg1, sg2, sg3, so0, so1, so2, so3,
    ):
        wid = lax.axis_index("s") * NC + lax.axis_index("c")
        b0 = wid * BLK_PER_W

        def gat_desc(t, buf, sem):
            return pltpu.make_async_copy(
                r_hbm.at[ixall.at[pl.ds(t * 128, 128)]], buf, sem
            )

        def out_desc(b, buf, sem):
            j = lax.shift_right_logical(b, 5)
            i_hi = lax.bitwise_and(b, 31)
            return pltpu.make_async_copy(
                buf, out_hbm.at[j, :, i_hi, :, :], sem
            )

        def transpose(rw, sl):
            i_vecs = [_iota16() + 16 * g for g in range(8)]
            rot = _iota16()

            @plsc.parallel_loop(0, D, unroll=8)
            def row(d):
                d_vec = lax.bitwise_and(rot + d, 63)
                d_hi = lax.shift_right_logical(d_vec, 3)
                d_lo = lax.bitwise_and(d_vec, 7)
                for g in range(8):
                    vals = plsc.load_gather(rw, [i_vecs[g], d_vec])
                    plsc.store_scatter(sl, [d_hi, d_lo, i_vecs[g]], vals)

        ixa_desc = pltpu.make_async_copy(
            xf_hbm.at[pl.ds(b0 * 128, BLK_PER_W * 128)], ixall, sxa
        )
        ixa_desc.start()
        ixa_desc.wait()
        bufs = (
            (0, rw0, sl0, sg0, so0),
            (1, rw1, sl1, sg1, so1),
            (2, rw2, sl2, sg2, so2),
            (3, rw3, sl3, sg3, so3),
        )
        for (k, rw, _, sg, _2) in bufs:
            gat_desc(k, rw, sg).start()

        def body(u, carry):
            for (k, rw, sl, sg, so) in bufs:
                t = 4 * u + k
                bb = b0 + t
                gat_desc(t, rw, sg).wait()
                pl.when(u > 0)(lambda: out_desc(bb - 4, sl, so).wait())
                transpose(rw, sl)
                out_desc(bb, sl, so).start()
                pl.when(u < BLK_PER_W // 4 - 1)(
                    lambda: gat_desc(t + 4, rw, sg).start()
                )

            return carry

        lax.fori_loop(0, BLK_PER_W // 4, body, 0)
        for (k, _, sl, _2, so) in bufs:
            out_desc(b0 + BLK_PER_W - 4 + k, sl, so).wait()

    return gather


def kernel(x, table):
    xf = x.T.reshape(B).astype(jnp.int32)
    tT = table.T
    repacked = _build_repack()(tT)
    r_lin = repacked.reshape(V, D)
    out5 = _build_gather()(xf, r_lin)
    return out5.transpose(2, 4, 0, 1, 3).reshape(4096, 200, D)


# speedup vs baseline: 9.9989x; 1.1700x over previous
import functools

import jax
import jax.numpy as jnp
from jax import lax
from jax.experimental import pallas as pl
from jax.experimental.pallas import tpu as pltpu
from jax.experimental.pallas import tpu_sc as plsc

V = 1_000_000
D = 64
VP = V // 2
NC, NS = 2, 16
NW = NC * NS
B = 4096 * 200
NBLK = B // 128
BLK_PER_W = NBLK // NW
FULL_COLS = (V // 128) * 128
NFULL2 = FULL_COLS // 256
RPW = (NFULL2 // NW) & ~1
NEXTRA = NFULL2 - RPW * NW


def _iota16():
    return lax.iota(jnp.int32, 16)


def _repack_block(in_v, out_v, nrows):
    rot = _iota16()
    r_vec = lax.shift_right_logical(rot, 1)
    h64_vec = lax.bitwise_and(rot, 1) * 64
    cols = [rot + 16 * q for q in range(nrows // 8)]
    rqs = [r_vec + 8 * q for q in range(nrows // 8)]

    @plsc.parallel_loop(0, D, unroll=4)
    def dim(d):
        d_vec = lax.bitwise_and(rot + d, 63)
        dh = d_vec + h64_vec
        for q in range(nrows // 8):
            vals = plsc.load_gather(in_v, [d_vec, cols[q]])
            plsc.store_scatter(out_v, [rqs[q], dh], vals)


def _build_repack():
    mesh = plsc.VectorSubcoreMesh(core_axis_name="c", subcore_axis_name="s")

    @functools.partial(
        pl.kernel,
        mesh=mesh,
        out_type=jax.ShapeDtypeStruct((VP, 128), jnp.float32),
        scratch_types=[
            pltpu.VMEM((D, 256), jnp.float32),
            pltpu.VMEM((D, 256), jnp.float32),
            pltpu.VMEM((D, 64), jnp.float32),
            pltpu.VMEM((128, 128), jnp.float32),
            pltpu.VMEM((128, 128), jnp.float32),
            pltpu.SemaphoreType.DMA,
            pltpu.SemaphoreType.DMA,
            pltpu.SemaphoreType.DMA,
            pltpu.SemaphoreType.DMA,
        ],
        compiler_params=pltpu.CompilerParams(use_tc_tiling_on_sc=True, needs_layout_passes=False),
    )
    def repack(tT_hbm, r_hbm, in0, in1, int_, o0, o1, si0, si1, so0, so1):
        wid = lax.axis_index("s") * NC + lax.axis_index("c")
        m0 = wid * RPW

        def in_desc(m, buf, sem):
            return pltpu.make_async_copy(
                tT_hbm.at[:, pl.ds(m * 256, 256)], buf, sem
            )

        def out_desc(m, buf, sem):
            return pltpu.make_async_copy(
                buf, r_hbm.at[pl.ds(m * 128, 128), :], sem
            )

        in_desc(m0, in0, si0).start()
        in_desc(m0 + 1, in1, si1).start()

        def body(u, carry):
            ma = m0 + 2 * u
            for (mb, in_v, out_v, si, so) in (
                (ma, in0, o0, si0, so0),
                (ma + 1, in1, o1, si1, so1),
            ):
                in_desc(mb, in_v, si).wait()
                pl.when(u > 0)(lambda: out_desc(mb - 2, out_v, so).wait())
                _repack_block(in_v, out_v, 128)
                out_desc(mb, out_v, so).start()
                pl.when(u < RPW // 2 - 1)(
                    lambda: in_desc(mb + 2, in_v, si).start()
                )
            return carry

        lax.fori_loop(0, RPW // 2, body, 0)
        out_desc(m0 + RPW - 2, o0, so0).wait()
        out_desc(m0 + RPW - 1, o1, so1).wait()

        @pl.when(wid < NEXTRA)
        def _extra():
            m = NFULL2 - NEXTRA + wid
            in_desc(m, in0, si0).start()
            in_desc(m, in0, si0).wait()
            _repack_block(in0, o0, 128)
            out_desc(m, o0, so0).start()
            out_desc(m, o0, so0).wait()

        @pl.when(wid == NW - 1)
        def _tail():
            tin = pltpu.make_async_copy(
                tT_hbm.at[:, pl.ds(FULL_COLS, 64)], int_, si1
            )
            tin.start()
            tin.wait()
            _repack_block(int_, o1, 32)
            tout = pltpu.make_async_copy(
                o1.at[pl.ds(0, 32), :],
                r_hbm.at[pl.ds(FULL_COLS // 2, 32), :],
                so1,
            )
            tout.start()
            tout.wait()

    return repack


def _build_gather():
    mesh = plsc.VectorSubcoreMesh(core_axis_name="c", subcore_axis_name="s")

    @functools.partial(
        pl.kernel,
        mesh=mesh,
        out_type=jax.ShapeDtypeStruct((200, 8, 32, 8, 128), jnp.float32),
        scratch_types=[
            pltpu.VMEM((BLK_PER_W * 128,), jnp.int32),
            pltpu.VMEM((128, D), jnp.float32),
            pltpu.VMEM((128, D), jnp.float32),
            pltpu.VMEM((128, D), jnp.float32),
            pltpu.VMEM((128, D), jnp.float32),
            pltpu.VMEM((8, 8, 128), jnp.float32),
            pltpu.VMEM((8, 8, 128), jnp.float32),
            pltpu.VMEM((8, 8, 128), jnp.float32),
            pltpu.VMEM((8, 8, 128), jnp.float32),
            pltpu.SemaphoreType.DMA,
            pltpu.SemaphoreType.DMA,
            pltpu.SemaphoreType.DMA,
            pltpu.SemaphoreType.DMA,
            pltpu.SemaphoreType.DMA,
            pltpu.SemaphoreType.DMA,
            pltpu.SemaphoreType.DMA,
            pltpu.SemaphoreType.DMA,
            pltpu.SemaphoreType.DMA,
        ],
        compiler_params=pltpu.CompilerParams(use_tc_tiling_on_sc=False, needs_layout_passes=False),
    )
    def gather(
        xf_hbm, r_hbm, out_hbm,
        ixall, rw0, rw1, rw2, rw3, sl0, sl1, sl2, sl3,
        sxa, sg0, sg1, sg2, sg3, so0, so1, so2, so3,
    ):
        wid = lax.axis_index("s") * NC + lax.axis_index("c")
        b0 = wid * BLK_PER_W

        def gat_desc(t, buf, sem):
            return pltpu.make_async_copy(
                r_hbm.at[ixall.at[pl.ds(t * 128, 128)]], buf, sem
            )

        def out_desc(b, buf, sem):
            j = lax.shift_right_logical(b, 5)
            i_hi = lax.bitwise_and(b, 31)
            return pltpu.make_async_copy(
                buf, out_hbm.at[j, :, i_hi, :, :], sem
            )

        def transpose(rw, sl):
            i_vecs = [_iota16() + 16 * g for g in range(8)]
            rot = _iota16()

            @plsc.parallel_loop(0, D, unroll=8)
            def row(d):
                d_vec = lax.bitwise_and(rot + d, 63)
                d_hi = lax.shift_right_logical(d_vec, 3)
                d_lo = lax.bitwise_and(d_vec, 7)
                for g in range(8):
                    vals = plsc.load_gather(rw, [i_vecs[g], d_vec])
                    plsc.store_scatter(sl, [d_hi, d_lo, i_vecs[g]], vals)

        ixa_desc = pltpu.make_async_copy(
            xf_hbm.at[pl.ds(b0 * 128, BLK_PER_W * 128)], ixall, sxa
        )
        ixa_desc.start()
        ixa_desc.wait()
        bufs = (
            (0, rw0, sl0, sg0, so0),
            (1, rw1, sl1, sg1, so1),
            (2, rw2, sl2, sg2, so2),
            (3, rw3, sl3, sg3, so3),
        )
        for (k, rw, _, sg, _2) in bufs:
            gat_desc(k, rw, sg).start()

        def body(u, carry):
            for (k, rw, sl, sg, so) in bufs:
                t = 4 * u + k
                bb = b0 + t
                gat_desc(t, rw, sg).wait()
                pl.when(u > 0)(lambda: out_desc(bb - 4, sl, so).wait())
                transpose(rw, sl)
                out_desc(bb, sl, so).start()
                pl.when(u < BLK_PER_W // 4 - 1)(
                    lambda: gat_desc(t + 4, rw, sg).start()
                )

            return carry

        lax.fori_loop(0, BLK_PER_W // 4, body, 0)
        for (k, _, sl, _2, so) in bufs:
            out_desc(b0 + BLK_PER_W - 4 + k, sl, so).wait()

    return gather


def kernel(x, table):
    xf = x.T.reshape(B).astype(jnp.int32)
    tT = table.T
    repacked = _build_repack()(tT)
    r_lin = repacked.reshape(V, D)
    out5 = _build_gather()(xf, r_lin)
    return out5.transpose(2, 4, 0, 1, 3).reshape(4096, 200, D)
